# baseline (speedup 1.0000x reference)
import functools

import jax
import jax.numpy as jnp
from jax import lax
from jax.experimental import pallas as pl
from jax.experimental.pallas import tpu as pltpu
from jax.experimental.pallas import tpu_sc as plsc

NUM_USERS = 1000000
NUM_ITEMS = 100000
DIM = 64
DIMW = DIM // 2
BATCH = 16384

NC = 2
NS = 16
NW = NC * NS
B_PER_W = BATCH // NW
CHUNK = 128
NCHUNK = B_PER_W // CHUNK
LAM_W = 16

UQ_LOG = 18
IQ_LOG = 15
PBLK4 = 4096

_MASK_HI = -65536


def _log_sigmoid(x):
    t = jnp.exp(-jnp.abs(x))
    w = t / (t + 2.0)
    w2 = w * w
    poly = 1.0 + w2 * (1.0 / 3.0 + w2 * (1.0 / 5.0 + w2 * (1.0 / 7.0 + w2 * (1.0 / 9.0))))
    return jnp.minimum(x, 0.0) - 2.0 * w * poly


def _pack4(x0, x1, x2, x3, blo, bhi):
    xb = jnp.concatenate([x0, x1, x2, x3], axis=0).astype(jnp.bfloat16)
    dn = (((0,), (0,)), ((), ()))
    ylo = lax.dot_general(xb, blo[...], dn, preferred_element_type=jnp.float32)
    yhi = lax.dot_general(xb, bhi[...], dn, preferred_element_type=jnp.float32)
    lo = lax.bitcast_convert_type(ylo, jnp.uint32) >> 16
    hi = lax.bitcast_convert_type(yhi, jnp.uint32) & jnp.uint32(0xFFFF0000)
    return lax.bitcast_convert_type(lo | hi, jnp.int32)


def _tc_pack_body(u0, u1, u2, u3, a0, a1, a2, a3, blo, bhi, ou_ref, oa_ref):
    pid = pl.program_id(0)
    on_theta = pid < UBLKS
    xs = [jnp.where(on_theta, u[...], a[...])
          for u, a in ((u0, a0), (u1, a1), (u2, a2), (u3, a3))]
    packed = _pack4(*xs, blo, bhi)

    @pl.when(on_theta)
    def _():
        ou_ref[...] = packed

    @pl.when(jnp.logical_not(on_theta))
    def _():
        oa_ref[...] = packed


UBLKS = (1 << UQ_LOG) // PBLK4
IBLKS = (1 << IQ_LOG) // PBLK4


def _tc_pack(theta_t, alpha_t):
    last_u = (theta_t.shape[1] - 1) // PBLK4
    last_i = (alpha_t.shape[1] - 1) // PBLK4

    def uspec(qi):
        return pl.BlockSpec(
            (DIM, PBLK4),
            lambda i: (0, jnp.minimum(qi * UBLKS + jnp.minimum(i, UBLKS - 1), last_u)))

    def ispec(qi):
        return pl.BlockSpec(
            (DIM, PBLK4),
            lambda i: (0, jnp.minimum(
                qi * IBLKS + jnp.clip(i - UBLKS, 0, IBLKS - 1), last_i)))

    k = jnp.arange(4 * DIM)
    t = k & (DIM - 1)
    qq = k // DIM
    lanes = jnp.arange(4 * DIMW)
    l_lo = jnp.where(t < DIMW, qq * DIMW + t, -1)
    l_hi = jnp.where(t >= DIMW, qq * DIMW + (t - DIMW), -1)
    b_lo = (l_lo[:, None] == lanes[None, :]).astype(jnp.bfloat16)
    b_hi = (l_hi[:, None] == lanes[None, :]).astype(jnp.bfloat16)

    bspec = pl.BlockSpec((4 * DIM, 4 * DIMW), lambda i: (0, 0))
    return pl.pallas_call(
        _tc_pack_body,
        grid=(UBLKS + IBLKS,),
        in_specs=[uspec(0), uspec(1), uspec(2), uspec(3),
                  ispec(0), ispec(1), ispec(2), ispec(3), bspec, bspec],
        out_specs=[
            pl.BlockSpec((PBLK4, 4 * DIMW),
                         lambda i: (jnp.minimum(i, UBLKS - 1), 0)),
            pl.BlockSpec((PBLK4, 4 * DIMW),
                         lambda i: (jnp.clip(i - UBLKS, 0, IBLKS - 1), 0)),
        ],
        out_shape=[
            jax.ShapeDtypeStruct((1 << UQ_LOG, 4 * DIMW), jnp.int32),
            jax.ShapeDtypeStruct((1 << IQ_LOG, 4 * DIMW), jnp.int32),
        ],
    )(theta_t, theta_t, theta_t, theta_t,
      alpha_t, alpha_t, alpha_t, alpha_t, b_lo, b_hi)


def _sc_body(uidx_hbm, iidx_hbm, theta_hbm, alpha_hbm, lam_hbm, out_hbm,
             idx_u, idx_i, idx_ur, idx_ir, idx_hi, u_rows, a_rows, lam_rows,
             out_buf, sem0, sem1, sem2, sem3):
    c = lax.axis_index("c")
    s = lax.axis_index("s")
    wid = s * NC + c
    sems = [sem0, sem1, sem2, sem3]

    for j in range(NCHUNK):
        src = pl.ds(wid * B_PER_W + j * CHUNK, CHUNK)
        pltpu.sync_copy(uidx_hbm.at[src], idx_u.at[j])
        pltpu.sync_copy(iidx_hbm.at[src], idx_i.at[j])

    lane = lax.iota(jnp.int32, 16)

    def prep(k, _):
        ch = jnp.full((16,), k >> 3, jnp.int32)
        pos = jnp.full((16,), (k & 7) * 16, jnp.int32) + lane
        uv = plsc.load_gather(idx_u, [ch, pos])
        iv = plsc.load_gather(idx_i, [ch, pos])
        plsc.store_scatter(idx_ur, [ch, pos], uv & ((1 << UQ_LOG) - 1))
        plsc.store_scatter(idx_ir, [ch, pos], iv & ((1 << IQ_LOG) - 1))
        plsc.store_scatter(idx_hi, [ch, pos], iv >> 4)
        return 0

    lax.fori_loop(0, B_PER_W // 16, prep, 0)

    def issue(j):
        half = pl.ds((j & 1) * CHUNK, CHUNK)
        return [
            pltpu.async_copy(theta_hbm.at[idx_ur.at[j]], u_rows.at[half], sems[j]),
            pltpu.async_copy(alpha_hbm.at[idx_ir.at[j]], a_rows.at[half], sems[j]),
            pltpu.async_copy(lam_hbm.at[idx_hi.at[j]], lam_rows.at[pl.ds(j * CHUNK, CHUNK)], sems[j]),
        ]

    copies = [issue(0), issue(1), None, None]

    zero = jnp.zeros((16,), jnp.float32)

    def make_group(j):
        def group(g, _):
            ch = jnp.full((16,), j, jnp.int32)
            pos = jnp.full((16,), (g & 7) * 16, jnp.int32) + lane
            rows = jnp.full((16,), (j & 1) * CHUNK + (g & 7) * 16, jnp.int32) + lane
            uv = plsc.load_gather(idx_u, [ch, pos])
            iv = plsc.load_gather(idx_i, [ch, pos])
            ovu = (uv >> (UQ_LOG - 5)) & 96
            ovi = (iv >> (IQ_LOG - 5)) & 96

            def dstep(t, carry):
                a0, a1, a2, a3, du, da = carry
                accs = [a0, a1, a2, a3]
                for k in range(8):
                    wu = plsc.load_gather(u_rows, [rows, du + k if k else du])
                    wa = plsc.load_gather(a_rows, [rows, da + k if k else da])
                    u_lo = lax.bitcast_convert_type(wu << 16, jnp.float32)
                    a_lo = lax.bitcast_convert_type(wa << 16, jnp.float32)
                    u_hi = lax.bitcast_convert_type(wu & _MASK_HI, jnp.float32)
                    a_hi = lax.bitcast_convert_type(wa & _MASK_HI, jnp.float32)
                    accs[(2 * k) & 3] = accs[(2 * k) & 3] + u_lo * a_lo
                    accs[(2 * k + 1) & 3] = accs[(2 * k + 1) & 3] + u_hi * a_hi
                return (accs[0], accs[1], accs[2], accs[3], du + 8, da + 8)

            a0, a1, a2, a3, _, _ = lax.fori_loop(
                0, DIMW // 8, dstep, (zero, zero, zero, zero, ovu, ovi))
            acc = (a0 + a1) + (a2 + a3)
            lamv = plsc.load_gather(lam_rows, [jnp.full((16,), j * CHUNK, jnp.int32) + pos, iv & 15])
            out_buf[pl.ds(j * CHUNK + (g & 7) * 16, 16)] = _log_sigmoid(acc + lamv)
            return 0

        return group

    for j in range(NCHUNK):
        for cp in copies[j]:
            cp.wait()
        lax.fori_loop(0, CHUNK // 16, make_group(j), 0)
        if j + 2 < NCHUNK:
            copies[j + 2] = issue(j + 2)

    pltpu.sync_copy(out_buf, out_hbm.at[pl.ds(wid * B_PER_W, B_PER_W)])


@jax.jit
def _run(uidx, iidx, theta_user, alpha_item, lambda_item):
    theta_p, alpha_p = _tc_pack(theta_user.T, alpha_item.T)
    lam2d = lambda_item.reshape(NUM_ITEMS // LAM_W, LAM_W)
    mesh = plsc.VectorSubcoreMesh(core_axis_name="c", subcore_axis_name="s")
    f = functools.partial(
        pl.kernel,
        mesh=mesh,
        out_type=jax.ShapeDtypeStruct((BATCH,), jnp.float32),
        compiler_params=pltpu.CompilerParams(
            needs_layout_passes=False, use_tc_tiling_on_sc=False),
        scratch_types=[
            pltpu.VMEM((NCHUNK, CHUNK), jnp.int32),
            pltpu.VMEM((NCHUNK, CHUNK), jnp.int32),
            pltpu.VMEM((NCHUNK, CHUNK), jnp.int32),
            pltpu.VMEM((NCHUNK, CHUNK), jnp.int32),
            pltpu.VMEM((NCHUNK, CHUNK), jnp.int32),
            pltpu.VMEM((2 * CHUNK, 4 * DIMW), jnp.int32),
            pltpu.VMEM((2 * CHUNK, 4 * DIMW), jnp.int32),
            pltpu.VMEM((B_PER_W, LAM_W), jnp.float32),
            pltpu.VMEM((B_PER_W,), jnp.float32),
            pltpu.SemaphoreType.DMA,
            pltpu.SemaphoreType.DMA,
            pltpu.SemaphoreType.DMA,
            pltpu.SemaphoreType.DMA,
        ],
    )(_sc_body)
    return f(uidx, iidx, theta_p, alpha_p, lam2d)


def kernel(user_index, item_index, theta_user, alpha_item, lambda_item):
    uidx = user_index.astype(jnp.int32)
    iidx = item_index.astype(jnp.int32)
    return _run(uidx, iidx, theta_user, alpha_item, lambda_item)


# speedup vs baseline: 1.0697x; 1.0697x over previous
import functools

import jax
import jax.numpy as jnp
from jax import lax
from jax.experimental import pallas as pl
from jax.experimental.pallas import tpu as pltpu
from jax.experimental.pallas import tpu_sc as plsc

NUM_USERS = 1000000
NUM_ITEMS = 100000
DIM = 64
DIMW = DIM // 2
BATCH = 16384

NC = 2
NS = 16
NW = NC * NS
B_PER_W = BATCH // NW
CHUNK = 128
NCHUNK = B_PER_W // CHUNK
LAM_W = 16

UQ_LOG = 18
IQ_LOG = 15
PBLK4 = 4096

_MASK_HI = -65536


def _log_sigmoid(x):
    t = jnp.exp(-jnp.abs(x))
    w = t / (t + 2.0)
    w2 = w * w
    poly = 1.0 + w2 * (1.0 / 3.0 + w2 * (1.0 / 5.0 + w2 * (1.0 / 7.0 + w2 * (1.0 / 9.0))))
    return jnp.minimum(x, 0.0) - 2.0 * w * poly


def _pack4(x0, x1, x2, x3, blo, bhi):
    xb = jnp.concatenate([x0, x1, x2, x3], axis=0).astype(jnp.bfloat16)
    dn = (((0,), (0,)), ((), ()))
    ylo = lax.dot_general(xb, blo[...], dn, preferred_element_type=jnp.float32)
    yhi = lax.dot_general(xb, bhi[...], dn, preferred_element_type=jnp.float32)
    lo = lax.bitcast_convert_type(ylo, jnp.uint32) >> 16
    hi = lax.bitcast_convert_type(yhi, jnp.uint32) & jnp.uint32(0xFFFF0000)
    return lax.bitcast_convert_type(lo | hi, jnp.int32)


def _tc_pack_body(x0, x1, x2, x3, blo, bhi, o_ref):
    o_ref[...] = _pack4(x0[...], x1[...], x2[...], x3[...], blo, bhi)


def _tc_pack(x_t, qlog):
    q = 1 << qlog
    nblk = q // PBLK4
    last = (x_t.shape[1] - 1) // PBLK4

    def spec(qi):
        return pl.BlockSpec((DIM, PBLK4), lambda i: (0, jnp.minimum(qi * nblk + i, last)))

    k = jnp.arange(4 * DIM)
    t = k & (DIM - 1)
    qq = k // DIM
    lanes = jnp.arange(4 * DIMW)
    l_lo = jnp.where(t < DIMW, qq * DIMW + t, -1)
    l_hi = jnp.where(t >= DIMW, qq * DIMW + (t - DIMW), -1)
    b_lo = (l_lo[:, None] == lanes[None, :]).astype(jnp.bfloat16)
    b_hi = (l_hi[:, None] == lanes[None, :]).astype(jnp.bfloat16)

    bspec = pl.BlockSpec((4 * DIM, 4 * DIMW), lambda i: (0, 0))
    return pl.pallas_call(
        _tc_pack_body,
        grid=(nblk,),
        in_specs=[spec(0), spec(1), spec(2), spec(3), bspec, bspec],
        out_specs=pl.BlockSpec((PBLK4, 4 * DIMW), lambda i: (i, 0)),
        out_shape=jax.ShapeDtypeStruct((q, 4 * DIMW), jnp.int32),
    )(x_t, x_t, x_t, x_t, b_lo, b_hi)


def _sc_body(uidx_hbm, iidx_hbm, theta_hbm, alpha_hbm, lam_hbm, out_hbm,
             idx_u, idx_i, idx_ur, idx_ir, idx_hi, u_rows, a_rows, lam_rows,
             out_buf, sem0, sem1, sem2, sem3):
    c = lax.axis_index("c")
    s = lax.axis_index("s")
    wid = s * NC + c
    sems = [sem0, sem1, sem2, sem3]

    for j in range(NCHUNK):
        src = pl.ds(wid * B_PER_W + j * CHUNK, CHUNK)
        pltpu.sync_copy(uidx_hbm.at[src], idx_u.at[j])
        pltpu.sync_copy(iidx_hbm.at[src], idx_i.at[j])

    lane = lax.iota(jnp.int32, 16)

    def prep(k, _):
        ch = jnp.full((16,), k >> 3, jnp.int32)
        pos = jnp.full((16,), (k & 7) * 16, jnp.int32) + lane
        uv = plsc.load_gather(idx_u, [ch, pos])
        iv = plsc.load_gather(idx_i, [ch, pos])
        plsc.store_scatter(idx_ur, [ch, pos], uv & ((1 << UQ_LOG) - 1))
        plsc.store_scatter(idx_ir, [ch, pos], iv & ((1 << IQ_LOG) - 1))
        plsc.store_scatter(idx_hi, [ch, pos], iv >> 4)
        return 0

    lax.fori_loop(0, B_PER_W // 16, prep, 0)

    def issue(j):
        half = pl.ds((j & 1) * CHUNK, CHUNK)
        return [
            pltpu.async_copy(theta_hbm.at[idx_ur.at[j]], u_rows.at[half], sems[j]),
            pltpu.async_copy(alpha_hbm.at[idx_ir.at[j]], a_rows.at[half], sems[j]),
            pltpu.async_copy(lam_hbm.at[idx_hi.at[j]], lam_rows.at[pl.ds(j * CHUNK, CHUNK)], sems[j]),
        ]

    copies = [issue(0), issue(1), None, None]

    zero = jnp.zeros((16,), jnp.float32)

    def make_group(j):
        def group(g, _):
            ch = jnp.full((16,), j, jnp.int32)
            pos = jnp.full((16,), (g & 7) * 16, jnp.int32) + lane
            rows = jnp.full((16,), (j & 1) * CHUNK + (g & 7) * 16, jnp.int32) + lane
            uv = plsc.load_gather(idx_u, [ch, pos])
            iv = plsc.load_gather(idx_i, [ch, pos])
            ovu = (uv >> (UQ_LOG - 5)) & 96
            ovi = (iv >> (IQ_LOG - 5)) & 96

            def dstep(t, carry):
                a0, a1, a2, a3, du, da = carry
                accs = [a0, a1, a2, a3]
                for k in range(8):
                    wu = plsc.load_gather(u_rows, [rows, du + k if k else du])
                    wa = plsc.load_gather(a_rows, [rows, da + k if k else da])
                    u_lo = lax.bitcast_convert_type(wu << 16, jnp.float32)
                    a_lo = lax.bitcast_convert_type(wa << 16, jnp.float32)
                    u_hi = lax.bitcast_convert_type(wu & _MASK_HI, jnp.float32)
                    a_hi = lax.bitcast_convert_type(wa & _MASK_HI, jnp.float32)
                    accs[(2 * k) & 3] = accs[(2 * k) & 3] + u_lo * a_lo
                    accs[(2 * k + 1) & 3] = accs[(2 * k + 1) & 3] + u_hi * a_hi
                return (accs[0], accs[1], accs[2], accs[3], du + 8, da + 8)

            a0, a1, a2, a3, _, _ = lax.fori_loop(
                0, DIMW // 8, dstep, (zero, zero, zero, zero, ovu, ovi))
            acc = (a0 + a1) + (a2 + a3)
            lamv = plsc.load_gather(lam_rows, [jnp.full((16,), j * CHUNK, jnp.int32) + pos, iv & 15])
            out_buf[pl.ds(j * CHUNK + (g & 7) * 16, 16)] = _log_sigmoid(acc + lamv)
            return 0

        return group

    for j in range(NCHUNK):
        for cp in copies[j]:
            cp.wait()
        lax.fori_loop(0, CHUNK // 16, make_group(j), 0)
        if j + 2 < NCHUNK:
            copies[j + 2] = issue(j + 2)

    pltpu.sync_copy(out_buf, out_hbm.at[pl.ds(wid * B_PER_W, B_PER_W)])


@jax.jit
def _run(uidx, iidx, theta_user, alpha_item, lambda_item):
    theta_p = _tc_pack(theta_user.T, UQ_LOG)
    alpha_p = _tc_pack(alpha_item.T, IQ_LOG)
    lam2d = lambda_item.reshape(NUM_ITEMS // LAM_W, LAM_W)
    mesh = plsc.VectorSubcoreMesh(core_axis_name="c", subcore_axis_name="s")
    f = functools.partial(
        pl.kernel,
        mesh=mesh,
        out_type=jax.ShapeDtypeStruct((BATCH,), jnp.float32),
        compiler_params=pltpu.CompilerParams(
            needs_layout_passes=False, use_tc_tiling_on_sc=False),
        scratch_types=[
            pltpu.VMEM((NCHUNK, CHUNK), jnp.int32),
            pltpu.VMEM((NCHUNK, CHUNK), jnp.int32),
            pltpu.VMEM((NCHUNK, CHUNK), jnp.int32),
            pltpu.VMEM((NCHUNK, CHUNK), jnp.int32),
            pltpu.VMEM((NCHUNK, CHUNK), jnp.int32),
            pltpu.VMEM((2 * CHUNK, 4 * DIMW), jnp.int32),
            pltpu.VMEM((2 * CHUNK, 4 * DIMW), jnp.int32),
            pltpu.VMEM((B_PER_W, LAM_W), jnp.float32),
            pltpu.VMEM((B_PER_W,), jnp.float32),
            pltpu.SemaphoreType.DMA,
            pltpu.SemaphoreType.DMA,
            pltpu.SemaphoreType.DMA,
            pltpu.SemaphoreType.DMA,
        ],
    )(_sc_body)
    return f(uidx, iidx, theta_p, alpha_p, lam2d)


def kernel(user_index, item_index, theta_user, alpha_item, lambda_item):
    uidx = user_index.astype(jnp.int32)
    iidx = item_index.astype(jnp.int32)
    return _run(uidx, iidx, theta_user, alpha_item, lambda_item)


# speedup vs baseline: 1.0854x; 1.0147x over previous
import functools

import jax
import jax.numpy as jnp
from jax import lax
from jax.experimental import pallas as pl
from jax.experimental.pallas import tpu as pltpu
from jax.experimental.pallas import tpu_sc as plsc

NUM_USERS = 1000000
NUM_ITEMS = 100000
DIM = 64
DIMW = DIM // 2
BATCH = 16384

NC = 2
NS = 16
NW = NC * NS
B_PER_W = BATCH // NW
CHUNK = 128
NCHUNK = B_PER_W // CHUNK
LAM_W = 16

UQ_LOG = 18
IQ_LOG = 15
PBLK4 = 4096

_MASK_HI = -65536


def _log_sigmoid(x):
    t = jnp.exp(-jnp.abs(x))
    w = t / (t + 2.0)
    w2 = w * w
    poly = 1.0 + w2 * (1.0 / 3.0 + w2 * (1.0 / 5.0 + w2 * (1.0 / 7.0 + w2 * (1.0 / 9.0))))
    return jnp.minimum(x, 0.0) - 2.0 * w * poly


def _pack4(x0, x1, x2, x3, blo, bhi):
    xb = jnp.concatenate([x0, x1, x2, x3], axis=0).astype(jnp.bfloat16)
    dn = (((0,), (0,)), ((), ()))
    ylo = lax.dot_general(xb, blo[...], dn, preferred_element_type=jnp.float32)
    yhi = lax.dot_general(xb, bhi[...], dn, preferred_element_type=jnp.float32)
    lo = lax.bitcast_convert_type(ylo, jnp.uint32) >> 16
    hi = lax.bitcast_convert_type(yhi, jnp.uint32) & jnp.uint32(0xFFFF0000)
    return lax.bitcast_convert_type(lo | hi, jnp.int32)


def _tc_pack_body(x0, x1, x2, x3, blo, bhi, o_ref):
    o_ref[...] = _pack4(x0[...], x1[...], x2[...], x3[...], blo, bhi)


def _tc_pack(x_t, qlog):
    q = 1 << qlog
    nblk = q // PBLK4
    last = (x_t.shape[1] - 1) // PBLK4

    def spec(qi):
        return pl.BlockSpec((DIM, PBLK4), lambda i: (0, jnp.minimum(qi * nblk + i, last)))

    k = jnp.arange(4 * DIM)
    t = k & (DIM - 1)
    qq = k // DIM
    lanes = jnp.arange(4 * DIMW)
    l_lo = jnp.where(t < DIMW, qq * DIMW + t, -1)
    l_hi = jnp.where(t >= DIMW, qq * DIMW + (t - DIMW), -1)
    b_lo = (l_lo[:, None] == lanes[None, :]).astype(jnp.bfloat16)
    b_hi = (l_hi[:, None] == lanes[None, :]).astype(jnp.bfloat16)

    bspec = pl.BlockSpec((4 * DIM, 4 * DIMW), lambda i: (0, 0))
    return pl.pallas_call(
        _tc_pack_body,
        grid=(nblk,),
        in_specs=[spec(0), spec(1), spec(2), spec(3), bspec, bspec],
        out_specs=pl.BlockSpec((PBLK4, 4 * DIMW), lambda i: (i, 0)),
        out_shape=jax.ShapeDtypeStruct((q, 4 * DIMW), jnp.int32),
    )(x_t, x_t, x_t, x_t, b_lo, b_hi)


def _sc_body(uidx_hbm, iidx_hbm, theta_hbm, alpha_hbm, lam_hbm, out_hbm,
             idx_u, idx_i, idx_ur, idx_ir, idx_hi, u_rows, a_rows, lam_rows,
             out_buf, sem0, sem1, sem2, sem3):
    c = lax.axis_index("c")
    s = lax.axis_index("s")
    wid = s * NC + c
    sems = [sem0, sem1, sem2, sem3]

    pltpu.sync_copy(uidx_hbm.at[wid], idx_u)
    pltpu.sync_copy(iidx_hbm.at[wid], idx_i)

    lane = lax.iota(jnp.int32, 16)

    def prep(k, _):
        ch = jnp.full((16,), k >> 3, jnp.int32)
        pos = jnp.full((16,), (k & 7) * 16, jnp.int32) + lane
        uv = plsc.load_gather(idx_u, [ch, pos])
        iv = plsc.load_gather(idx_i, [ch, pos])
        plsc.store_scatter(idx_ur, [ch, pos], uv & ((1 << UQ_LOG) - 1))
        plsc.store_scatter(idx_ir, [ch, pos], iv & ((1 << IQ_LOG) - 1))
        plsc.store_scatter(idx_hi, [ch, pos], iv >> 4)
        return 0

    lax.fori_loop(0, B_PER_W // 16, prep, 0)

    def issue(j):
        half = pl.ds((j & 1) * CHUNK, CHUNK)
        return [
            pltpu.async_copy(theta_hbm.at[idx_ur.at[j]], u_rows.at[half], sems[j]),
            pltpu.async_copy(alpha_hbm.at[idx_ir.at[j]], a_rows.at[half], sems[j]),
            pltpu.async_copy(lam_hbm.at[idx_hi.at[j]], lam_rows.at[pl.ds(j * CHUNK, CHUNK)], sems[j]),
        ]

    copies = [issue(0), issue(1), None, None]

    zero = jnp.zeros((16,), jnp.float32)

    def make_group(j):
        def group(g, _):
            ch = jnp.full((16,), j, jnp.int32)
            pos = jnp.full((16,), (g & 7) * 16, jnp.int32) + lane
            rows = jnp.full((16,), (j & 1) * CHUNK + (g & 7) * 16, jnp.int32) + lane
            uv = plsc.load_gather(idx_u, [ch, pos])
            iv = plsc.load_gather(idx_i, [ch, pos])
            ovu = (uv >> (UQ_LOG - 5)) & 96
            ovi = (iv >> (IQ_LOG - 5)) & 96

            def dstep(t, carry):
                a0, a1, a2, a3, du, da = carry
                accs = [a0, a1, a2, a3]
                for k in range(8):
                    wu = plsc.load_gather(u_rows, [rows, du + k if k else du])
                    wa = plsc.load_gather(a_rows, [rows, da + k if k else da])
                    u_lo = lax.bitcast_convert_type(wu << 16, jnp.float32)
                    a_lo = lax.bitcast_convert_type(wa << 16, jnp.float32)
                    u_hi = lax.bitcast_convert_type(wu & _MASK_HI, jnp.float32)
                    a_hi = lax.bitcast_convert_type(wa & _MASK_HI, jnp.float32)
                    accs[(2 * k) & 3] = accs[(2 * k) & 3] + u_lo * a_lo
                    accs[(2 * k + 1) & 3] = accs[(2 * k + 1) & 3] + u_hi * a_hi
                return (accs[0], accs[1], accs[2], accs[3], du + 8, da + 8)

            a0, a1, a2, a3, _, _ = lax.fori_loop(
                0, DIMW // 8, dstep, (zero, zero, zero, zero, ovu, ovi))
            acc = (a0 + a1) + (a2 + a3)
            lamv = plsc.load_gather(lam_rows, [jnp.full((16,), j * CHUNK, jnp.int32) + pos, iv & 15])
            out_buf[pl.ds(j * CHUNK + (g & 7) * 16, 16)] = _log_sigmoid(acc + lamv)
            return 0

        return group

    for j in range(NCHUNK):
        for cp in copies[j]:
            cp.wait()
        lax.fori_loop(0, CHUNK // 16, make_group(j), 0)
        if j + 2 < NCHUNK:
            copies[j + 2] = issue(j + 2)

    pltpu.sync_copy(out_buf, out_hbm.at[pl.ds(wid * B_PER_W, B_PER_W)])


@jax.jit
def _run(uidx, iidx, theta_user, alpha_item, lambda_item):
    theta_p = _tc_pack(theta_user.T, UQ_LOG)
    alpha_p = _tc_pack(alpha_item.T, IQ_LOG)
    lam2d = lambda_item.reshape(NUM_ITEMS // LAM_W, LAM_W)
    mesh = plsc.VectorSubcoreMesh(core_axis_name="c", subcore_axis_name="s")
    f = functools.partial(
        pl.kernel,
        mesh=mesh,
        out_type=jax.ShapeDtypeStruct((BATCH,), jnp.float32),
        compiler_params=pltpu.CompilerParams(
            needs_layout_passes=False, use_tc_tiling_on_sc=False),
        scratch_types=[
            pltpu.VMEM((NCHUNK, CHUNK), jnp.int32),
            pltpu.VMEM((NCHUNK, CHUNK), jnp.int32),
            pltpu.VMEM((NCHUNK, CHUNK), jnp.int32),
            pltpu.VMEM((NCHUNK, CHUNK), jnp.int32),
            pltpu.VMEM((NCHUNK, CHUNK), jnp.int32),
            pltpu.VMEM((2 * CHUNK, 4 * DIMW), jnp.int32),
            pltpu.VMEM((2 * CHUNK, 4 * DIMW), jnp.int32),
            pltpu.VMEM((B_PER_W, LAM_W), jnp.float32),
            pltpu.VMEM((B_PER_W,), jnp.float32),
            pltpu.SemaphoreType.DMA,
            pltpu.SemaphoreType.DMA,
            pltpu.SemaphoreType.DMA,
            pltpu.SemaphoreType.DMA,
        ],
    )(_sc_body)
    return f(uidx, iidx, theta_p, alpha_p, lam2d)


def kernel(user_index, item_index, theta_user, alpha_item, lambda_item):
    uidx = user_index.astype(jnp.int32).reshape(NW, NCHUNK, CHUNK)
    iidx = item_index.astype(jnp.int32).reshape(NW, NCHUNK, CHUNK)
    return _run(uidx, iidx, theta_user, alpha_item, lambda_item)


# speedup vs baseline: 1.1664x; 1.0746x over previous
import functools

import jax
import jax.numpy as jnp
from jax import lax
from jax.experimental import pallas as pl
from jax.experimental.pallas import tpu as pltpu
from jax.experimental.pallas import tpu_sc as plsc

NUM_USERS = 1000000
NUM_ITEMS = 100000
DIM = 64
DIMW = DIM // 2
BATCH = 16384

NC = 2
NS = 16
NW = NC * NS
B_PER_W = BATCH // NW
CHUNK = 128
NCHUNK = B_PER_W // CHUNK
LAM_W = 16

UQ_LOG = 18
IQ_LOG = 15
PBLK4 = 8192

_MASK_HI = -65536


def _log_sigmoid(x):
    t = jnp.exp(-jnp.abs(x))
    w = t / (t + 2.0)
    w2 = w * w
    poly = 1.0 + w2 * (1.0 / 3.0 + w2 * (1.0 / 5.0 + w2 * (1.0 / 7.0 + w2 * (1.0 / 9.0))))
    return jnp.minimum(x, 0.0) - 2.0 * w * poly


def _pack4(x0, x1, x2, x3, blo, bhi):
    xb = jnp.concatenate([x0, x1, x2, x3], axis=0).astype(jnp.bfloat16)
    dn = (((0,), (0,)), ((), ()))
    ylo = lax.dot_general(xb, blo[...], dn, preferred_element_type=jnp.float32)
    yhi = lax.dot_general(xb, bhi[...], dn, preferred_element_type=jnp.float32)
    lo = lax.bitcast_convert_type(ylo, jnp.uint32) >> 16
    hi = lax.bitcast_convert_type(yhi, jnp.uint32) & jnp.uint32(0xFFFF0000)
    return lax.bitcast_convert_type(lo | hi, jnp.int32)


def _tc_pack_body(x0, x1, x2, x3, blo, bhi, o_ref):
    o_ref[...] = _pack4(x0[...], x1[...], x2[...], x3[...], blo, bhi)


def _tc_pack(x_t, qlog):
    q = 1 << qlog
    nblk = q // PBLK4
    last = (x_t.shape[1] - 1) // PBLK4

    def spec(qi):
        return pl.BlockSpec((DIM, PBLK4), lambda i: (0, jnp.minimum(qi * nblk + i, last)))

    k = jnp.arange(4 * DIM)
    t = k & (DIM - 1)
    qq = k // DIM
    lanes = jnp.arange(4 * DIMW)
    l_lo = jnp.where(t < DIMW, qq * DIMW + t, -1)
    l_hi = jnp.where(t >= DIMW, qq * DIMW + (t - DIMW), -1)
    b_lo = (l_lo[:, None] == lanes[None, :]).astype(jnp.bfloat16)
    b_hi = (l_hi[:, None] == lanes[None, :]).astype(jnp.bfloat16)

    bspec = pl.BlockSpec((4 * DIM, 4 * DIMW), lambda i: (0, 0))
    return pl.pallas_call(
        _tc_pack_body,
        grid=(nblk,),
        in_specs=[spec(0), spec(1), spec(2), spec(3), bspec, bspec],
        out_specs=pl.BlockSpec((PBLK4, 4 * DIMW), lambda i: (i, 0)),
        out_shape=jax.ShapeDtypeStruct((q, 4 * DIMW), jnp.int32),
    )(x_t, x_t, x_t, x_t, b_lo, b_hi)


def _sc_body(uidx_hbm, iidx_hbm, theta_hbm, alpha_hbm, lam_hbm, out_hbm,
             idx_u, idx_i, idx_ur, idx_ir, idx_hi, u_rows, a_rows, lam_rows,
             out_buf, sem0, sem1, sem2, sem3):
    c = lax.axis_index("c")
    s = lax.axis_index("s")
    wid = s * NC + c
    sems = [sem0, sem1, sem2, sem3]

    pltpu.sync_copy(uidx_hbm.at[wid], idx_u)
    pltpu.sync_copy(iidx_hbm.at[wid], idx_i)

    lane = lax.iota(jnp.int32, 16)

    def prep(k, _):
        ch = jnp.full((16,), k >> 3, jnp.int32)
        pos = jnp.full((16,), (k & 7) * 16, jnp.int32) + lane
        uv = plsc.load_gather(idx_u, [ch, pos])
        iv = plsc.load_gather(idx_i, [ch, pos])
        plsc.store_scatter(idx_ur, [ch, pos], uv & ((1 << UQ_LOG) - 1))
        plsc.store_scatter(idx_ir, [ch, pos], iv & ((1 << IQ_LOG) - 1))
        plsc.store_scatter(idx_hi, [ch, pos], iv >> 4)
        return 0

    lax.fori_loop(0, B_PER_W // 16, prep, 0)

    def issue(j):
        half = pl.ds((j & 1) * CHUNK, CHUNK)
        return [
            pltpu.async_copy(theta_hbm.at[idx_ur.at[j]], u_rows.at[half], sems[j]),
            pltpu.async_copy(alpha_hbm.at[idx_ir.at[j]], a_rows.at[half], sems[j]),
            pltpu.async_copy(lam_hbm.at[idx_hi.at[j]], lam_rows.at[pl.ds(j * CHUNK, CHUNK)], sems[j]),
        ]

    copies = [issue(0), issue(1), None, None]

    zero = jnp.zeros((16,), jnp.float32)

    def make_group(j):
        def group(g, _):
            ch = jnp.full((16,), j, jnp.int32)
            pos = jnp.full((16,), (g & 7) * 16, jnp.int32) + lane
            rows = jnp.full((16,), (j & 1) * CHUNK + (g & 7) * 16, jnp.int32) + lane
            uv = plsc.load_gather(idx_u, [ch, pos])
            iv = plsc.load_gather(idx_i, [ch, pos])
            ovu = (uv >> (UQ_LOG - 5)) & 96
            ovi = (iv >> (IQ_LOG - 5)) & 96

            def dstep(t, carry):
                a0, a1, a2, a3, du, da = carry
                accs = [a0, a1, a2, a3]
                for k in range(8):
                    wu = plsc.load_gather(u_rows, [rows, du + k if k else du])
                    wa = plsc.load_gather(a_rows, [rows, da + k if k else da])
                    u_lo = lax.bitcast_convert_type(wu << 16, jnp.float32)
                    a_lo = lax.bitcast_convert_type(wa << 16, jnp.float32)
                    u_hi = lax.bitcast_convert_type(wu & _MASK_HI, jnp.float32)
                    a_hi = lax.bitcast_convert_type(wa & _MASK_HI, jnp.float32)
                    accs[(2 * k) & 3] = accs[(2 * k) & 3] + u_lo * a_lo
                    accs[(2 * k + 1) & 3] = accs[(2 * k + 1) & 3] + u_hi * a_hi
                return (accs[0], accs[1], accs[2], accs[3], du + 8, da + 8)

            a0, a1, a2, a3, _, _ = lax.fori_loop(
                0, DIMW // 8, dstep, (zero, zero, zero, zero, ovu, ovi))
            acc = (a0 + a1) + (a2 + a3)
            lamv = plsc.load_gather(lam_rows, [jnp.full((16,), j * CHUNK, jnp.int32) + pos, iv & 15])
            out_buf[pl.ds(j * CHUNK + (g & 7) * 16, 16)] = _log_sigmoid(acc + lamv)
            return 0

        return group

    for j in range(NCHUNK):
        for cp in copies[j]:
            cp.wait()
        lax.fori_loop(0, CHUNK // 16, make_group(j), 0)
        if j + 2 < NCHUNK:
            copies[j + 2] = issue(j + 2)

    pltpu.sync_copy(out_buf, out_hbm.at[pl.ds(wid * B_PER_W, B_PER_W)])


@jax.jit
def _run(uidx, iidx, theta_user, alpha_item, lambda_item):
    theta_p = _tc_pack(theta_user.T, UQ_LOG)
    alpha_p = _tc_pack(alpha_item.T, IQ_LOG)
    lam2d = lambda_item.reshape(NUM_ITEMS // LAM_W, LAM_W)
    mesh = plsc.VectorSubcoreMesh(core_axis_name="c", subcore_axis_name="s")
    f = functools.partial(
        pl.kernel,
        mesh=mesh,
        out_type=jax.ShapeDtypeStruct((BATCH,), jnp.float32),
        compiler_params=pltpu.CompilerParams(
            needs_layout_passes=False, use_tc_tiling_on_sc=False),
        scratch_types=[
            pltpu.VMEM((NCHUNK, CHUNK), jnp.int32),
            pltpu.VMEM((NCHUNK, CHUNK), jnp.int32),
            pltpu.VMEM((NCHUNK, CHUNK), jnp.int32),
            pltpu.VMEM((NCHUNK, CHUNK), jnp.int32),
            pltpu.VMEM((NCHUNK, CHUNK), jnp.int32),
            pltpu.VMEM((2 * CHUNK, 4 * DIMW), jnp.int32),
            pltpu.VMEM((2 * CHUNK, 4 * DIMW), jnp.int32),
            pltpu.VMEM((B_PER_W, LAM_W), jnp.float32),
            pltpu.VMEM((B_PER_W,), jnp.float32),
            pltpu.SemaphoreType.DMA,
            pltpu.SemaphoreType.DMA,
            pltpu.SemaphoreType.DMA,
            pltpu.SemaphoreType.DMA,
        ],
    )(_sc_body)
    return f(uidx, iidx, theta_p, alpha_p, lam2d)


def kernel(user_index, item_index, theta_user, alpha_item, lambda_item):
    uidx = user_index.astype(jnp.int32).reshape(NW, NCHUNK, CHUNK)
    iidx = item_index.astype(jnp.int32).reshape(NW, NCHUNK, CHUNK)
    return _run(uidx, iidx, theta_user, alpha_item, lambda_item)


# speedup vs baseline: 1.1837x; 1.0148x over previous
import functools

import jax
import jax.numpy as jnp
from jax import lax
from jax.experimental import pallas as pl
from jax.experimental.pallas import tpu as pltpu
from jax.experimental.pallas import tpu_sc as plsc

NUM_USERS = 1000000
NUM_ITEMS = 100000
DIM = 64
DIMW = DIM // 2
BATCH = 16384

NC = 2
NS = 16
NW = NC * NS
B_PER_W = BATCH // NW
CHUNK = 128
NCHUNK = B_PER_W // CHUNK
LAM_W = 16

UQ_LOG = 18
IQ_LOG = 15
PBLK4 = 16384

_MASK_HI = -65536


def _log_sigmoid(x):
    t = jnp.exp(-jnp.abs(x))
    w = t / (t + 2.0)
    w2 = w * w
    poly = 1.0 + w2 * (1.0 / 3.0 + w2 * (1.0 / 5.0 + w2 * (1.0 / 7.0 + w2 * (1.0 / 9.0))))
    return jnp.minimum(x, 0.0) - 2.0 * w * poly


def _pack4(x0, x1, x2, x3, blo, bhi):
    xb = jnp.concatenate([x0, x1, x2, x3], axis=0).astype(jnp.bfloat16)
    dn = (((0,), (0,)), ((), ()))
    ylo = lax.dot_general(xb, blo[...], dn, preferred_element_type=jnp.float32)
    yhi = lax.dot_general(xb, bhi[...], dn, preferred_element_type=jnp.float32)
    lo = lax.bitcast_convert_type(ylo, jnp.uint32) >> 16
    hi = lax.bitcast_convert_type(yhi, jnp.uint32) & jnp.uint32(0xFFFF0000)
    return lax.bitcast_convert_type(lo | hi, jnp.int32)


def _tc_pack_body(x0, x1, x2, x3, blo, bhi, o_ref):
    o_ref[...] = _pack4(x0[...], x1[...], x2[...], x3[...], blo, bhi)


def _tc_pack(x_t, qlog):
    q = 1 << qlog
    nblk = q // PBLK4
    last = (x_t.shape[1] - 1) // PBLK4

    def spec(qi):
        return pl.BlockSpec((DIM, PBLK4), lambda i: (0, jnp.minimum(qi * nblk + i, last)))

    k = jnp.arange(4 * DIM)
    t = k & (DIM - 1)
    qq = k // DIM
    lanes = jnp.arange(4 * DIMW)
    l_lo = jnp.where(t < DIMW, qq * DIMW + t, -1)
    l_hi = jnp.where(t >= DIMW, qq * DIMW + (t - DIMW), -1)
    b_lo = (l_lo[:, None] == lanes[None, :]).astype(jnp.bfloat16)
    b_hi = (l_hi[:, None] == lanes[None, :]).astype(jnp.bfloat16)

    bspec = pl.BlockSpec((4 * DIM, 4 * DIMW), lambda i: (0, 0))
    return pl.pallas_call(
        _tc_pack_body,
        grid=(nblk,),
        in_specs=[spec(0), spec(1), spec(2), spec(3), bspec, bspec],
        out_specs=pl.BlockSpec((PBLK4, 4 * DIMW), lambda i: (i, 0)),
        out_shape=jax.ShapeDtypeStruct((q, 4 * DIMW), jnp.int32),
    )(x_t, x_t, x_t, x_t, b_lo, b_hi)


def _sc_body(uidx_hbm, iidx_hbm, theta_hbm, alpha_hbm, lam_hbm, out_hbm,
             idx_u, idx_i, idx_ur, idx_ir, idx_hi, u_rows, a_rows, lam_rows,
             out_buf, sem0, sem1, sem2, sem3):
    c = lax.axis_index("c")
    s = lax.axis_index("s")
    wid = s * NC + c
    sems = [sem0, sem1, sem2, sem3]

    pltpu.sync_copy(uidx_hbm.at[wid], idx_u)
    pltpu.sync_copy(iidx_hbm.at[wid], idx_i)

    lane = lax.iota(jnp.int32, 16)

    def prep(k, _):
        ch = jnp.full((16,), k >> 3, jnp.int32)
        pos = jnp.full((16,), (k & 7) * 16, jnp.int32) + lane
        uv = plsc.load_gather(idx_u, [ch, pos])
        iv = plsc.load_gather(idx_i, [ch, pos])
        plsc.store_scatter(idx_ur, [ch, pos], uv & ((1 << UQ_LOG) - 1))
        plsc.store_scatter(idx_ir, [ch, pos], iv & ((1 << IQ_LOG) - 1))
        plsc.store_scatter(idx_hi, [ch, pos], iv >> 4)
        return 0

    lax.fori_loop(0, B_PER_W // 16, prep, 0)

    def issue(j):
        half = pl.ds((j & 1) * CHUNK, CHUNK)
        return [
            pltpu.async_copy(theta_hbm.at[idx_ur.at[j]], u_rows.at[half], sems[j]),
            pltpu.async_copy(alpha_hbm.at[idx_ir.at[j]], a_rows.at[half], sems[j]),
            pltpu.async_copy(lam_hbm.at[idx_hi.at[j]], lam_rows.at[pl.ds(j * CHUNK, CHUNK)], sems[j]),
        ]

    copies = [issue(0), issue(1), None, None]

    zero = jnp.zeros((16,), jnp.float32)

    def make_group(j):
        def group(g, _):
            ch = jnp.full((16,), j, jnp.int32)
            pos = jnp.full((16,), (g & 7) * 16, jnp.int32) + lane
            rows = jnp.full((16,), (j & 1) * CHUNK + (g & 7) * 16, jnp.int32) + lane
            uv = plsc.load_gather(idx_u, [ch, pos])
            iv = plsc.load_gather(idx_i, [ch, pos])
            ovu = (uv >> (UQ_LOG - 5)) & 96
            ovi = (iv >> (IQ_LOG - 5)) & 96

            def dstep(t, carry):
                a0, a1, a2, a3, du, da = carry
                accs = [a0, a1, a2, a3]
                for k in range(8):
                    wu = plsc.load_gather(u_rows, [rows, du + k if k else du])
                    wa = plsc.load_gather(a_rows, [rows, da + k if k else da])
                    u_lo = lax.bitcast_convert_type(wu << 16, jnp.float32)
                    a_lo = lax.bitcast_convert_type(wa << 16, jnp.float32)
                    u_hi = lax.bitcast_convert_type(wu & _MASK_HI, jnp.float32)
                    a_hi = lax.bitcast_convert_type(wa & _MASK_HI, jnp.float32)
                    accs[(2 * k) & 3] = accs[(2 * k) & 3] + u_lo * a_lo
                    accs[(2 * k + 1) & 3] = accs[(2 * k + 1) & 3] + u_hi * a_hi
                return (accs[0], accs[1], accs[2], accs[3], du + 8, da + 8)

            a0, a1, a2, a3, _, _ = lax.fori_loop(
                0, DIMW // 8, dstep, (zero, zero, zero, zero, ovu, ovi))
            acc = (a0 + a1) + (a2 + a3)
            lamv = plsc.load_gather(lam_rows, [jnp.full((16,), j * CHUNK, jnp.int32) + pos, iv & 15])
            out_buf[pl.ds(j * CHUNK + (g & 7) * 16, 16)] = _log_sigmoid(acc + lamv)
            return 0

        return group

    for j in range(NCHUNK):
        for cp in copies[j]:
            cp.wait()
        lax.fori_loop(0, CHUNK // 16, make_group(j), 0)
        if j + 2 < NCHUNK:
            copies[j + 2] = issue(j + 2)

    pltpu.sync_copy(out_buf, out_hbm.at[pl.ds(wid * B_PER_W, B_PER_W)])


@jax.jit
def _run(uidx, iidx, theta_user, alpha_item, lambda_item):
    theta_p = _tc_pack(theta_user.T, UQ_LOG)
    alpha_p = _tc_pack(alpha_item.T, IQ_LOG)
    lam2d = lambda_item.reshape(NUM_ITEMS // LAM_W, LAM_W)
    mesh = plsc.VectorSubcoreMesh(core_axis_name="c", subcore_axis_name="s")
    f = functools.partial(
        pl.kernel,
        mesh=mesh,
        out_type=jax.ShapeDtypeStruct((BATCH,), jnp.float32),
        compiler_params=pltpu.CompilerParams(
            needs_layout_passes=False, use_tc_tiling_on_sc=False),
        scratch_types=[
            pltpu.VMEM((NCHUNK, CHUNK), jnp.int32),
            pltpu.VMEM((NCHUNK, CHUNK), jnp.int32),
            pltpu.VMEM((NCHUNK, CHUNK), jnp.int32),
            pltpu.VMEM((NCHUNK, CHUNK), jnp.int32),
            pltpu.VMEM((NCHUNK, CHUNK), jnp.int32),
            pltpu.VMEM((2 * CHUNK, 4 * DIMW), jnp.int32),
            pltpu.VMEM((2 * CHUNK, 4 * DIMW), jnp.int32),
            pltpu.VMEM((B_PER_W, LAM_W), jnp.float32),
            pltpu.VMEM((B_PER_W,), jnp.float32),
            pltpu.SemaphoreType.DMA,
            pltpu.SemaphoreType.DMA,
            pltpu.SemaphoreType.DMA,
            pltpu.SemaphoreType.DMA,
        ],
    )(_sc_body)
    return f(uidx, iidx, theta_p, alpha_p, lam2d)


def kernel(user_index, item_index, theta_user, alpha_item, lambda_item):
    uidx = user_index.astype(jnp.int32).reshape(NW, NCHUNK, CHUNK)
    iidx = item_index.astype(jnp.int32).reshape(NW, NCHUNK, CHUNK)
    return _run(uidx, iidx, theta_user, alpha_item, lambda_item)
